# baseline (speedup 1.0000x reference)
import functools

import jax
import jax.numpy as jnp
from jax import lax
from jax.experimental import pallas as pl
from jax.experimental.pallas import tpu as pltpu
from jax.experimental.pallas import tpu_sc as plsc

_NS = 16
_L = 16
_D = 16
_N = 65536
_CH = _N // _NS
_NT = 7813
_TPW = 489
_G = 8
_QS = _NT * _L + 16
_GI = 128
_SCALE = float(1000000)


def _body(anno_hbm, table_hbm, pos_hbm, out_hbm, scr_hbm,
          anno_v, idx_v, q_v, tb_v, ob_v, land_v, rows_v, pos_v, sem):
    wid = lax.axis_index("s")

    iota = lax.iota(jnp.int32, _L)
    ones = jnp.ones((_L,), jnp.float32)
    qsub = iota >> 3
    lsub = (iota & 7) * _L

    tmax = jnp.minimum((wid + 1) * _TPW, _NT) - 1

    def det_body(g, carry):
        t0 = wid * _TPW + g * _G
        ts = [jnp.minimum(t0 + b, tmax) for b in range(_G)]
        copies = []
        for b in range(_G):
            off = pl.multiple_of(ts[b] * 128, 128)
            copies.append(pltpu.async_copy(
                table_hbm.at[pl.ds(0, 8), pl.ds(off, 128)],
                tb_v.at[pl.ds(0, 8), pl.ds(b * 128, 128)], sem))
            copies.append(pltpu.async_copy(
                table_hbm.at[pl.ds(8, 8), pl.ds(off, 128)],
                tb_v.at[pl.ds(8, 8), pl.ds(b * 128, 128)], sem))
        for c in copies:
            c.wait()

        for b in range(_G):
            for k0 in range(0, 128, _L):
                qq = b * _L + k0 // 8 + qsub
                for c in range(16):
                    vals = tb_v[c, pl.ds(b * 128 + k0, _L)]
                    plsc.store_scatter(ob_v, [qq, lsub + c], vals)

        ocopies = []
        for b in range(_G):
            qoff = pl.multiple_of(ts[b] * _L, _L)
            ocopies.append(pltpu.async_copy(
                ob_v.at[pl.ds(b * _L, _L)],
                scr_hbm.at[pl.ds(qoff, _L)], sem))
        for c in ocopies:
            c.wait()
        return carry

    lax.fori_loop(0, (_TPW + _G - 1) // _G, det_body, 0)

    plsc.subcore_barrier()

    base = wid * _CH
    pltpu.sync_copy(anno_hbm.at[pl.ds(base, _CH)], anno_v)
    pltpu.sync_copy(pos_hbm, pos_v)

    def idx_body(k, carry):
        for m in range(8):
            off = k * 128 + m * _L
            x = anno_v[pl.ds(off, _L)]
            x = jnp.minimum(jnp.maximum(x, 0.0), 1.0)
            iv = (x * _SCALE).astype(jnp.int32)
            idx_v[pl.ds(off, _L)] = iv
            q_v[k, pl.ds(m * _L, _L)] = iv >> 3
        return carry

    lax.fori_loop(0, _CH // _GI, idx_body, 0)

    def grp_body(gg, carry):
        pltpu.async_copy(scr_hbm.at[q_v.at[gg]], land_v, sem).wait()
        for blk in range(_GI // _L):
            rlo = idx_v[pl.ds(gg * _GI + blk * _L, _L)] & 7
            lrow = blk * _L + iota
            cbase = rlo * _L
            cols = []
            acc = jnp.zeros((_L,), jnp.float32)
            for j in range(_D):
                c = plsc.load_gather(land_v, [lrow, cbase + j])
                cols.append(c)
                acc = acc + c * c
            yi = jnp.int32(0x5F3759DF) - (plsc.bitcast(acc, jnp.int32) >> 1)
            y = plsc.bitcast(yi, jnp.float32)
            for _ in range(3):
                y = y * (1.5 - 0.5 * acc * y * y)
            scale = jnp.where(acc > 1.0, y, ones)
            qrow = 2 * (gg * (_GI // _L) + blk) + qsub
            for j in range(_D):
                o = cols[j] * scale + pos_v[pl.ds(j * _L, _L)]
                plsc.store_scatter(rows_v, [qrow, lsub + j], o)
        return carry

    lax.fori_loop(0, _CH // _GI, grp_body, 0)

    pltpu.sync_copy(rows_v, out_hbm.at[pl.ds(wid * (_CH * _D // 128),
                                             _CH * _D // 128)])


_emb_lookup = functools.partial(
    pl.kernel,
    out_type=(
        jax.ShapeDtypeStruct((_N * _D // 128, 128), jnp.float32),
        jax.ShapeDtypeStruct((_QS, 128), jnp.float32),
    ),
    mesh=plsc.VectorSubcoreMesh(
        core_axis_name="c", subcore_axis_name="s", num_cores=1
    ),
    scratch_types=[
        pltpu.VMEM((_CH,), jnp.float32),
        pltpu.VMEM((_CH,), jnp.int32),
        pltpu.VMEM((_CH // _GI, _GI), jnp.int32),
        pltpu.VMEM((16, _G * 128), jnp.float32),
        pltpu.VMEM((_G * _L, 128), jnp.float32),
        pltpu.VMEM((_GI, 128), jnp.float32),
        pltpu.VMEM((_CH * _D // 128, 128), jnp.float32),
        pltpu.VMEM((_D * _L,), jnp.float32),
        pltpu.SemaphoreType.DMA,
    ],
    compiler_params=pltpu.CompilerParams(
        needs_layout_passes=False, use_tc_tiling_on_sc=True
    ),
)(_body)


@jax.jit
def kernel(past_search_anno, table, pos_embed):
    b, s = past_search_anno.shape
    anno_flat = past_search_anno.reshape(-1)
    table_t = table.T
    pos_t = jnp.tile(pos_embed[0], (_L // s, 1)).T.reshape(-1)
    out, _ = _emb_lookup(anno_flat, table_t, pos_t)
    return out.reshape(b, s, _D)


# speedup vs baseline: 1.1548x; 1.1548x over previous
import functools

import jax
import jax.numpy as jnp
from jax import lax
from jax.experimental import pallas as pl
from jax.experimental.pallas import tpu as pltpu
from jax.experimental.pallas import tpu_sc as plsc

_NS = 16
_L = 16
_D = 16
_N = 65536
_CH = _N // _NS
_NT = 7813
_TPW = 489
_G = 4
_NG = 124
_QS = _NT * _L + 64
_GI = 128
_SCALE = float(1000000)


def _body(anno_hbm, table_hbm, pos_hbm, out_hbm, scr_hbm,
          anno_v, idx_v, q_v, tba_v, tbb_v, oba_v, obb_v, land_v, rows_v,
          pos_v, sem, osem):
    wid = lax.axis_index("s")

    iota = lax.iota(jnp.int32, _L)
    ones = jnp.ones((_L,), jnp.float32)
    qsub = iota >> 3
    lsub = (iota & 7) * _L

    tmax = jnp.minimum((wid + 1) * _TPW, _NT) - 1
    g0 = wid * _TPW

    def t_of(grp):
        return jnp.minimum(g0 + grp * _G, tmax - (_G - 1))

    def fire_in(grp, tb):
        t0c = t_of(grp)
        for b in range(_G):
            off = pl.multiple_of((t0c + b) * 128, 128)
            pltpu.async_copy(
                table_hbm.at[pl.ds(0, 8), pl.ds(off, 128)],
                tb.at[pl.ds(0, 8), pl.ds(b * 128, 128)], sem)
            pltpu.async_copy(
                table_hbm.at[pl.ds(8, 8), pl.ds(off, 128)],
                tb.at[pl.ds(8, 8), pl.ds(b * 128, 128)], sem)

    def drain_in(tb):
        for b in range(_G):
            pltpu.make_async_copy(
                table_hbm.at[pl.ds(0, 8), pl.ds(0, 128)],
                tb.at[pl.ds(0, 8), pl.ds(b * 128, 128)], sem).wait()
            pltpu.make_async_copy(
                table_hbm.at[pl.ds(8, 8), pl.ds(0, 128)],
                tb.at[pl.ds(8, 8), pl.ds(b * 128, 128)], sem).wait()

    def drain_out(ob):
        pltpu.make_async_copy(scr_hbm.at[pl.ds(0, _G * _L)], ob, osem).wait()

    def assemble(grp, tb, ob):
        for b in range(_G):
            for k0 in range(0, 128, _L):
                qq = b * _L + k0 // 8 + qsub
                for c in range(16):
                    vals = tb[c, pl.ds(b * 128 + k0, _L)]
                    plsc.store_scatter(ob, [qq, lsub + c], vals)
        qoff = pl.multiple_of(t_of(grp) * _L, _L)
        pltpu.async_copy(ob, scr_hbm.at[pl.ds(qoff, _G * _L)], osem)

    fire_in(0, tba_v)
    pltpu.async_copy(oba_v, scr_hbm.at[pl.ds(_NT * _L, _G * _L)], osem)
    pltpu.async_copy(obb_v, scr_hbm.at[pl.ds(_NT * _L, _G * _L)], osem)

    def det_body(g, carry):
        fire_in(2 * g + 1, tbb_v)
        drain_in(tba_v)
        drain_out(oba_v)
        assemble(2 * g, tba_v, oba_v)
        fire_in(2 * g + 2, tba_v)
        drain_in(tbb_v)
        drain_out(obb_v)
        assemble(2 * g + 1, tbb_v, obb_v)
        return carry

    lax.fori_loop(0, _NG // 2, det_body, 0)
    drain_in(tba_v)
    drain_out(oba_v)
    drain_out(obb_v)

    plsc.subcore_barrier()

    base = wid * _CH
    pltpu.sync_copy(anno_hbm.at[pl.ds(base, _CH)], anno_v)
    pltpu.sync_copy(pos_hbm, pos_v)

    def idx_body(k, carry):
        for m in range(8):
            off = k * 128 + m * _L
            x = anno_v[pl.ds(off, _L)]
            x = jnp.minimum(jnp.maximum(x, 0.0), 1.0)
            iv = (x * _SCALE).astype(jnp.int32)
            idx_v[pl.ds(off, _L)] = iv
            q_v[k, pl.ds(m * _L, _L)] = iv >> 3
        return carry

    lax.fori_loop(0, _CH // _GI, idx_body, 0)

    obase = wid * (_CH * _D // 128)

    def grp_body(gg, carry):
        pltpu.async_copy(scr_hbm.at[q_v.at[gg]], land_v, sem).wait()
        for blk in range(_GI // _L):
            rlo = idx_v[pl.ds(gg * _GI + blk * _L, _L)] & 7
            lrow = blk * _L + iota
            cbase = rlo * _L
            cols = []
            acc = jnp.zeros((_L,), jnp.float32)
            for j in range(_D):
                c = plsc.load_gather(land_v, [lrow, cbase + j])
                cols.append(c)
                acc = acc + c * c
            yi = jnp.int32(0x5F3759DF) - (plsc.bitcast(acc, jnp.int32) >> 1)
            y = plsc.bitcast(yi, jnp.float32)
            for _ in range(3):
                y = y * (1.5 - 0.5 * acc * y * y)
            scale = jnp.where(acc > 1.0, y, ones)
            qrow = 2 * blk + qsub
            for j in range(_D):
                o = cols[j] * scale + pos_v[pl.ds(j * _L, _L)]
                plsc.store_scatter(rows_v, [qrow, lsub + j], o)
        pltpu.sync_copy(rows_v, out_hbm.at[pl.ds(obase + gg * _L, _L)])
        return carry

    lax.fori_loop(0, _CH // _GI, grp_body, 0)


_emb_lookup = functools.partial(
    pl.kernel,
    out_type=(
        jax.ShapeDtypeStruct((_N * _D // 128, 128), jnp.float32),
        jax.ShapeDtypeStruct((_QS, 128), jnp.float32),
    ),
    mesh=plsc.VectorSubcoreMesh(
        core_axis_name="c", subcore_axis_name="s", num_cores=1
    ),
    scratch_types=[
        pltpu.VMEM((_CH,), jnp.float32),
        pltpu.VMEM((_CH,), jnp.int32),
        pltpu.VMEM((_CH // _GI, _GI), jnp.int32),
        pltpu.VMEM((16, _G * 128), jnp.float32),
        pltpu.VMEM((16, _G * 128), jnp.float32),
        pltpu.VMEM((_G * _L, 128), jnp.float32),
        pltpu.VMEM((_G * _L, 128), jnp.float32),
        pltpu.VMEM((_GI, 128), jnp.float32),
        pltpu.VMEM((_L, 128), jnp.float32),
        pltpu.VMEM((_D * _L,), jnp.float32),
        pltpu.SemaphoreType.DMA,
        pltpu.SemaphoreType.DMA,
    ],
    compiler_params=pltpu.CompilerParams(
        needs_layout_passes=False, use_tc_tiling_on_sc=True
    ),
)(_body)


@jax.jit
def kernel(past_search_anno, table, pos_embed):
    b, s = past_search_anno.shape
    anno_flat = past_search_anno.reshape(-1)
    table_t = table.T
    pos_t = jnp.tile(pos_embed[0], (_L // s, 1)).T.reshape(-1)
    out, _ = _emb_lookup(anno_flat, table_t, pos_t)
    return out.reshape(b, s, _D)
